# baseline (speedup 1.0000x reference)
import functools

import jax
import jax.numpy as jnp
from jax import lax
from jax.experimental import pallas as pl
from jax.experimental.pallas import tpu as pltpu
from jax.experimental.pallas import tpu_sc as plsc

VOCAB = 256
N_BLOCKS = 4
BLOCK_ROWS = 64
DIM = 64
B, L = 1024, 20
N_TOK = B * L
PADW = 2 * DIM

_info = plsc.get_sparse_core_info()
_NC, _NS, _LANES = _info.num_cores, _info.num_subcores, _info.num_lanes
_NW = _NC * _NS
_TOK_PER_W = N_TOK // _NW


def _make_sc_kernel():
    mesh = plsc.VectorSubcoreMesh(core_axis_name="c", subcore_axis_name="s")

    @functools.partial(
        pl.kernel,
        mesh=mesh,
        out_type=jax.ShapeDtypeStruct((N_TOK, PADW), jnp.float32),
        scratch_types=[
            pltpu.VMEM((_TOK_PER_W,), jnp.int32),
            pltpu.VMEM((_TOK_PER_W, PADW), jnp.float32),
            pltpu.SemaphoreType.DMA,
        ],
    )
    def sc_kernel(src_hbm, table_hbm, out_hbm, idx_v, rows_v, sem):
        wid = lax.axis_index("s") * _NC + lax.axis_index("c")
        base = wid * _TOK_PER_W
        pltpu.sync_copy(src_hbm.at[pl.ds(base, _TOK_PER_W)], idx_v)
        pltpu.async_copy(table_hbm.at[idx_v], rows_v, sem).wait()
        pltpu.sync_copy(rows_v, out_hbm.at[pl.ds(base, _TOK_PER_W)])

    return sc_kernel


_sc_kernel = _make_sc_kernel()


def kernel(src, block_assign, local_assign, W0, W1, W2, W3):
    table = jnp.concatenate([W0, W1, W2, W3], axis=0)
    row_map = block_assign * BLOCK_ROWS + local_assign
    table = table.at[row_map].get(mode="promise_in_bounds", unique_indices=True)
    table = jnp.pad(table, ((0, 0), (0, PADW - DIM)))
    flat_src = src.reshape(N_TOK)
    out = _sc_kernel(flat_src, table)
    return out[:, :DIM].reshape(B, L, DIM)


# speedup vs baseline: 1.1373x; 1.1373x over previous
import functools

import jax
import jax.numpy as jnp
from jax import lax
from jax.experimental import pallas as pl
from jax.experimental.pallas import tpu as pltpu
from jax.experimental.pallas import tpu_sc as plsc

VOCAB = 256
N_BLOCKS = 4
BLOCK_ROWS = 64
DIM = 64
B, L = 1024, 20
N_TOK = B * L

_info = plsc.get_sparse_core_info()
_NC, _NS, _LANES = _info.num_cores, _info.num_subcores, _info.num_lanes
_NW = _NC * _NS


def _make_sc_kernel(n_tok):
    tok_per_w = n_tok // _NW
    mesh = plsc.VectorSubcoreMesh(core_axis_name="c", subcore_axis_name="s")

    @functools.partial(
        pl.kernel,
        mesh=mesh,
        out_type=jax.ShapeDtypeStruct((n_tok, DIM), jnp.float32),
        compiler_params=pltpu.CompilerParams(use_tc_tiling_on_sc=False),
        scratch_types=[
            pltpu.VMEM((tok_per_w,), jnp.int32),
            pltpu.VMEM((tok_per_w, DIM), jnp.float32),
            pltpu.SemaphoreType.DMA,
        ],
    )
    def sc_kernel(src_hbm, table_hbm, out_hbm, idx_v, rows_v, sem):
        wid = lax.axis_index("s") * _NC + lax.axis_index("c")
        base = wid * tok_per_w
        pltpu.sync_copy(src_hbm.at[pl.ds(base, tok_per_w)], idx_v)
        pltpu.async_copy(table_hbm.at[idx_v], rows_v, sem).wait()
        pltpu.sync_copy(rows_v, out_hbm.at[pl.ds(base, tok_per_w)])

    return sc_kernel


_sc_half = _make_sc_kernel(N_TOK // 2)


def kernel(src, block_assign, local_assign, W0, W1, W2, W3):
    table = jnp.concatenate([W0, W1, W2, W3], axis=0)
    row_map = block_assign * BLOCK_ROWS + local_assign
    table = table.at[row_map].get(mode="promise_in_bounds", unique_indices=True)
    flat_src = src.reshape(N_TOK)
    half = N_TOK // 2
    out_a = _sc_half(flat_src[:half], table)
    out_b = _sc_half(flat_src[half:], table)
    out = jnp.concatenate(
        [out_a.reshape(B // 2, L, DIM), out_b.reshape(B // 2, L, DIM)], axis=0)
    return out
